# baseline (speedup 1.0000x reference)
import jax
import jax.numpy as jnp
from jax import lax
from jax.experimental import pallas as pl
from jax.experimental.pallas import tpu as pltpu
from jax.experimental.pallas import tpu_sc as plsc

N = 100000
D = 128
V = 512
B = 80
NBLK = N // B
NC = 2
NS = 16
NW = NC * NS
ROUNDS = (NBLK + NW - 1) // NW
LANES = 16
CHUNKS = D // LANES
VSLICE = V // NS


def _body(x_hbm, din_hbm, dout_hbm, tin_hbm, tout_hbm, out_hbm,
          tin_sp, tout_sp,
          xb0, ab0, bb0, ii0, io0, xb1, ab1, bb1, ii1, io1,
          is0, xs0, gs0, os0, is1, xs1, gs1, os1):
    cid = lax.axis_index("c")
    sid = lax.axis_index("s")
    w = sid * NC + cid

    def stage_table(t_hbm, t_sp, tmp, sem):
        rows = pl.ds(sid * VSLICE, VSLICE)
        pltpu.async_copy(t_hbm.at[rows], tmp, sem).wait()
        pltpu.sync_copy(tmp, t_sp.at[rows])

    stage_table(tin_hbm, tin_sp, ab0.at[pl.ds(0, VSLICE)], gs0)
    stage_table(tout_hbm, tout_sp, bb0.at[pl.ds(0, VSLICE)], gs0)
    plsc.subcore_barrier()

    slot0 = (xb0, ab0, bb0, ii0, io0, is0, xs0, gs0, os0)
    slot1 = (xb1, ab1, bb1, ii1, io1, is1, xs1, gs1, os1)

    def active(r):
        return (r >= 0) & (r < ROUNDS) & (r * NW + w < NBLK)

    def stage(r, s):
        xb, ab, bb, ii, io, isem, xsem, gsem, osem = s
        base = (r * NW + w) * B
        pltpu.async_copy(din_hbm.at[pl.ds(base, B)], ii, isem)
        pltpu.async_copy(dout_hbm.at[pl.ds(base, B)], io, isem)
        pltpu.async_copy(x_hbm.at[pl.ds(base, B)], xb, xsem)

    def wait_idx_issue_gathers(s):
        xb, ab, bb, ii, io, isem, xsem, gsem, osem = s
        pltpu.make_async_copy(din_hbm.at[pl.ds(0, B)], ii, isem).wait()
        pltpu.make_async_copy(dout_hbm.at[pl.ds(0, B)], io, isem).wait()
        pltpu.async_copy(tin_sp.at[ii], ab, gsem)
        pltpu.async_copy(tout_sp.at[io], bb, gsem)

    def wait_loads(s):
        xb, ab, bb, ii, io, isem, xsem, gsem, osem = s
        pltpu.make_async_copy(x_hbm.at[pl.ds(0, B)], xb, xsem).wait()
        pltpu.make_async_copy(tin_sp.at[pl.ds(0, B)], ab, gsem).wait()
        pltpu.make_async_copy(tout_sp.at[pl.ds(0, B)], bb, gsem).wait()

    def compute_and_scatter(r, s):
        xb, ab, bb, ii, io, isem, xsem, gsem, osem = s

        @plsc.parallel_loop(0, B, 1, unroll=4)
        def row_body(i):
            for cc in range(CHUNKS):
                sl = pl.ds(cc * LANES, LANES)
                xb[i, sl] = xb[i, sl] + ab[i, sl] + bb[i, sl]
        base = (r * NW + w) * B
        pltpu.async_copy(xb, out_hbm.at[pl.ds(base, B)], osem)

    def wait_scatter(s):
        xb, ab, bb, ii, io, isem, xsem, gsem, osem = s
        pltpu.make_async_copy(xb, out_hbm.at[pl.ds(0, B)], osem).wait()

    def emit_round(r, cur, nxt):
        @pl.when(active(r - 1))
        def _():
            wait_scatter(nxt)

        @pl.when(active(r + 1))
        def _():
            stage(r + 1, nxt)
            wait_idx_issue_gathers(nxt)

        @pl.when(active(r))
        def _():
            wait_loads(cur)
            compute_and_scatter(r, cur)

    @pl.when(active(0))
    def _():
        stage(0, slot0)
        wait_idx_issue_gathers(slot0)

    def pair_body(g, carry):
        emit_round(2 * g, slot0, slot1)
        emit_round(2 * g + 1, slot1, slot0)
        return carry

    lax.fori_loop(0, ROUNDS // 2, pair_body, 0)

    @pl.when(active(ROUNDS - 1))
    def _():
        wait_scatter(slot1)


@jax.jit
def kernel(x, in_degrees, out_degrees, in_degree_table, out_degree_table):
    mesh = plsc.VectorSubcoreMesh(
        core_axis_name="c", subcore_axis_name="s",
        num_cores=NC, num_subcores=NS,
    )
    xbuf = lambda: pltpu.VMEM((B, D), jnp.float32)
    ibuf = lambda: pltpu.VMEM((B,), jnp.int32)
    f = pl.kernel(
        _body,
        out_type=jax.ShapeDtypeStruct((N, D), jnp.float32),
        mesh=mesh,
        scratch_types=[
            pltpu.VMEM_SHARED((V, D), jnp.float32),
            pltpu.VMEM_SHARED((V, D), jnp.float32),
            xbuf(), xbuf(), xbuf(), ibuf(), ibuf(),
            xbuf(), xbuf(), xbuf(), ibuf(), ibuf(),
            pltpu.SemaphoreType.DMA, pltpu.SemaphoreType.DMA,
            pltpu.SemaphoreType.DMA, pltpu.SemaphoreType.DMA,
            pltpu.SemaphoreType.DMA, pltpu.SemaphoreType.DMA,
            pltpu.SemaphoreType.DMA, pltpu.SemaphoreType.DMA,
        ],
    )
    return f(x, in_degrees.astype(jnp.int32), out_degrees.astype(jnp.int32),
             in_degree_table, out_degree_table)


# speedup vs baseline: 1.0772x; 1.0772x over previous
import jax
import jax.numpy as jnp
from jax import lax
from jax.experimental import pallas as pl
from jax.experimental.pallas import tpu as pltpu
from jax.experimental.pallas import tpu_sc as plsc

N = 100000
D = 128
V = 512
B = 128
NBLK = (N + B - 1) // B
MAXBASE = N - B
NC = 2
NS = 16
NW = NC * NS
ROUNDS = (NBLK + NW - 1) // NW
LANES = 16
CHUNKS = D // LANES
VSLICE = V // NS


def _body(x_hbm, din_hbm, dout_hbm, tin_hbm, tout_hbm, out_hbm,
          tin_sp, tout_sp,
          xb0, ab0, bb0, ii0, io0, xb1, ab1, bb1, ii1, io1,
          is0, xs0, gs0, os0, is1, xs1, gs1, os1):
    cid = lax.axis_index("c")
    sid = lax.axis_index("s")
    w = sid * NC + cid

    def stage_table(t_hbm, t_sp, tmp, sem):
        rows = pl.ds(sid * VSLICE, VSLICE)
        pltpu.async_copy(t_hbm.at[rows], tmp, sem).wait()
        pltpu.sync_copy(tmp, t_sp.at[rows])

    stage_table(tin_hbm, tin_sp, ab0.at[pl.ds(0, VSLICE)], gs0)
    stage_table(tout_hbm, tout_sp, bb0.at[pl.ds(0, VSLICE)], gs0)
    plsc.subcore_barrier()

    slot0 = (xb0, ab0, bb0, ii0, io0, is0, xs0, gs0, os0)
    slot1 = (xb1, ab1, bb1, ii1, io1, is1, xs1, gs1, os1)

    def active(r):
        return (r >= 0) & (r < ROUNDS) & (r * NW + w < NBLK)

    def block_base(r):
        return jnp.minimum((r * NW + w) * B, MAXBASE)

    def stage(r, s):
        xb, ab, bb, ii, io, isem, xsem, gsem, osem = s
        base = block_base(r)
        pltpu.async_copy(din_hbm.at[pl.ds(base, B)], ii, isem)
        pltpu.async_copy(dout_hbm.at[pl.ds(base, B)], io, isem)
        pltpu.async_copy(x_hbm.at[pl.ds(base, B)], xb, xsem)

    def wait_idx_issue_gathers(s):
        xb, ab, bb, ii, io, isem, xsem, gsem, osem = s
        pltpu.make_async_copy(din_hbm.at[pl.ds(0, B)], ii, isem).wait()
        pltpu.make_async_copy(dout_hbm.at[pl.ds(0, B)], io, isem).wait()
        pltpu.async_copy(tin_sp.at[ii], ab, gsem)
        pltpu.async_copy(tout_sp.at[io], bb, gsem)

    def wait_loads(s):
        xb, ab, bb, ii, io, isem, xsem, gsem, osem = s
        pltpu.make_async_copy(x_hbm.at[pl.ds(0, B)], xb, xsem).wait()
        pltpu.make_async_copy(tin_sp.at[pl.ds(0, B)], ab, gsem).wait()
        pltpu.make_async_copy(tout_sp.at[pl.ds(0, B)], bb, gsem).wait()

    def compute_and_scatter(r, s):
        xb, ab, bb, ii, io, isem, xsem, gsem, osem = s

        def row_body(i, c):
            for cc in range(CHUNKS):
                sl = pl.ds(cc * LANES, LANES)
                xb[i, sl] = xb[i, sl] + ab[i, sl] + bb[i, sl]
            return c

        lax.fori_loop(0, B, row_body, 0)
        pltpu.async_copy(xb, out_hbm.at[pl.ds(block_base(r), B)], osem)

    def wait_scatter(s):
        xb, ab, bb, ii, io, isem, xsem, gsem, osem = s
        pltpu.make_async_copy(xb, out_hbm.at[pl.ds(0, B)], osem).wait()

    def emit_round(r, cur, nxt):
        @pl.when(active(r - 1))
        def _():
            wait_scatter(nxt)

        @pl.when(active(r + 1))
        def _():
            stage(r + 1, nxt)
            wait_idx_issue_gathers(nxt)

        @pl.when(active(r))
        def _():
            wait_loads(cur)
            compute_and_scatter(r, cur)

    @pl.when(active(0))
    def _():
        stage(0, slot0)
        wait_idx_issue_gathers(slot0)

    def pair_body(g, carry):
        emit_round(2 * g, slot0, slot1)
        emit_round(2 * g + 1, slot1, slot0)
        return carry

    lax.fori_loop(0, (ROUNDS + 1) // 2, pair_body, 0)


@jax.jit
def kernel(x, in_degrees, out_degrees, in_degree_table, out_degree_table):
    mesh = plsc.VectorSubcoreMesh(
        core_axis_name="c", subcore_axis_name="s",
        num_cores=NC, num_subcores=NS,
    )
    xbuf = lambda: pltpu.VMEM((B, D), jnp.float32)
    ibuf = lambda: pltpu.VMEM((B,), jnp.int32)
    f = pl.kernel(
        _body,
        out_type=jax.ShapeDtypeStruct((N, D), jnp.float32),
        mesh=mesh,
        scratch_types=[
            pltpu.VMEM_SHARED((V, D), jnp.float32),
            pltpu.VMEM_SHARED((V, D), jnp.float32),
            xbuf(), xbuf(), xbuf(), ibuf(), ibuf(),
            xbuf(), xbuf(), xbuf(), ibuf(), ibuf(),
            pltpu.SemaphoreType.DMA, pltpu.SemaphoreType.DMA,
            pltpu.SemaphoreType.DMA, pltpu.SemaphoreType.DMA,
            pltpu.SemaphoreType.DMA, pltpu.SemaphoreType.DMA,
            pltpu.SemaphoreType.DMA, pltpu.SemaphoreType.DMA,
        ],
    )
    return f(x, in_degrees.astype(jnp.int32), out_degrees.astype(jnp.int32),
             in_degree_table, out_degree_table)


# speedup vs baseline: 1.2684x; 1.1775x over previous
import jax
import jax.numpy as jnp
from jax import lax
from jax.experimental import pallas as pl
from jax.experimental.pallas import tpu as pltpu
from jax.experimental.pallas import tpu_sc as plsc

N = 100000
D = 128
V = 512
B = 128
NBLK = (N + B - 1) // B
MAXBASE = N - B
NC = 2
NS = 16
NW = NC * NS
ROUNDS = (NBLK + NW - 1) // NW
LANES = 16
CHUNKS = D // LANES
VSLICE = V // NS


def _body(x_hbm, din_hbm, dout_hbm, tin_hbm, tout_hbm, out_hbm,
          tin_sp, tout_sp,
          xb0, ab0, bb0, ii0, io0, xb1, ab1, bb1, ii1, io1,
          is0, xs0, gs0, os0, is1, xs1, gs1, os1):
    cid = lax.axis_index("c")
    sid = lax.axis_index("s")
    w = sid * NC + cid

    def stage_table(t_hbm, t_sp, tmp, sem):
        rows = pl.ds(sid * VSLICE, VSLICE)
        pltpu.async_copy(t_hbm.at[rows], tmp, sem).wait()
        pltpu.sync_copy(tmp, t_sp.at[rows])

    stage_table(tin_hbm, tin_sp, ab0.at[pl.ds(0, VSLICE)], gs0)
    stage_table(tout_hbm, tout_sp, bb0.at[pl.ds(0, VSLICE)], gs0)
    plsc.subcore_barrier()

    slot0 = (xb0, ab0, bb0, ii0, io0, is0, xs0, gs0, os0)
    slot1 = (xb1, ab1, bb1, ii1, io1, is1, xs1, gs1, os1)

    def active(r):
        return (r >= 0) & (r < ROUNDS) & (r * NW + w < NBLK)

    def block_base(r):
        return jnp.minimum((r * NW + w) * B, MAXBASE)

    def stage_idx(r, s):
        xb, ab, bb, ii, io, isem, xsem, gsem, osem = s
        base = block_base(r)
        pltpu.async_copy(din_hbm.at[pl.ds(base, B)], ii, isem)
        pltpu.async_copy(dout_hbm.at[pl.ds(base, B)], io, isem)

    def stage_x(r, s):
        xb, ab, bb, ii, io, isem, xsem, gsem, osem = s
        pltpu.async_copy(x_hbm.at[pl.ds(block_base(r), B)], xb, xsem)

    def wait_idx_issue_gathers(s):
        xb, ab, bb, ii, io, isem, xsem, gsem, osem = s
        pltpu.make_async_copy(din_hbm.at[pl.ds(0, B)], ii, isem).wait()
        pltpu.make_async_copy(dout_hbm.at[pl.ds(0, B)], io, isem).wait()
        pltpu.async_copy(tin_sp.at[ii], ab, gsem)
        pltpu.async_copy(tout_sp.at[io], bb, gsem)

    def wait_loads(s):
        xb, ab, bb, ii, io, isem, xsem, gsem, osem = s
        pltpu.make_async_copy(x_hbm.at[pl.ds(0, B)], xb, xsem).wait()
        pltpu.make_async_copy(tin_sp.at[pl.ds(0, B)], ab, gsem).wait()
        pltpu.make_async_copy(tout_sp.at[pl.ds(0, B)], bb, gsem).wait()

    def compute_and_scatter(r, s):
        xb, ab, bb, ii, io, isem, xsem, gsem, osem = s

        def row_body(i, c):
            for cc in range(CHUNKS):
                sl = pl.ds(cc * LANES, LANES)
                xb[i, sl] = xb[i, sl] + ab[i, sl] + bb[i, sl]
            return c

        lax.fori_loop(0, B, row_body, 0)
        pltpu.async_copy(xb, out_hbm.at[pl.ds(block_base(r), B)], osem)

    def wait_scatter(s):
        xb, ab, bb, ii, io, isem, xsem, gsem, osem = s
        pltpu.make_async_copy(xb, out_hbm.at[pl.ds(0, B)], osem).wait()

    def emit_round(r, cur, nxt):
        @pl.when(active(r - 1))
        def _():
            wait_scatter(nxt)

        @pl.when(active(r + 1))
        def _():
            stage_x(r + 1, nxt)
            wait_idx_issue_gathers(nxt)

        @pl.when(active(r))
        def _():
            wait_loads(cur)

        @pl.when(active(r + 2))
        def _():
            stage_idx(r + 2, cur)

        @pl.when(active(r))
        def _():
            compute_and_scatter(r, cur)

    @pl.when(active(0))
    def _():
        stage_idx(0, slot0)

    @pl.when(active(1))
    def _():
        stage_idx(1, slot1)

    @pl.when(active(0))
    def _():
        stage_x(0, slot0)
        wait_idx_issue_gathers(slot0)

    def pair_body(g, carry):
        emit_round(2 * g, slot0, slot1)
        emit_round(2 * g + 1, slot1, slot0)
        return carry

    lax.fori_loop(0, (ROUNDS + 1) // 2, pair_body, 0)


@jax.jit
def kernel(x, in_degrees, out_degrees, in_degree_table, out_degree_table):
    mesh = plsc.VectorSubcoreMesh(
        core_axis_name="c", subcore_axis_name="s",
        num_cores=NC, num_subcores=NS,
    )
    xbuf = lambda: pltpu.VMEM((B, D), jnp.float32)
    ibuf = lambda: pltpu.VMEM((B,), jnp.int32)
    f = pl.kernel(
        _body,
        out_type=jax.ShapeDtypeStruct((N, D), jnp.float32),
        mesh=mesh,
        scratch_types=[
            pltpu.VMEM_SHARED((V, D), jnp.float32),
            pltpu.VMEM_SHARED((V, D), jnp.float32),
            xbuf(), xbuf(), xbuf(), ibuf(), ibuf(),
            xbuf(), xbuf(), xbuf(), ibuf(), ibuf(),
            pltpu.SemaphoreType.DMA, pltpu.SemaphoreType.DMA,
            pltpu.SemaphoreType.DMA, pltpu.SemaphoreType.DMA,
            pltpu.SemaphoreType.DMA, pltpu.SemaphoreType.DMA,
            pltpu.SemaphoreType.DMA, pltpu.SemaphoreType.DMA,
        ],
    )
    return f(x, in_degrees.astype(jnp.int32), out_degrees.astype(jnp.int32),
             in_degree_table, out_degree_table)


# speedup vs baseline: 1.3775x; 1.0860x over previous
import jax
import jax.numpy as jnp
from jax import lax
from jax.experimental import pallas as pl
from jax.experimental.pallas import tpu as pltpu
from jax.experimental.pallas import tpu_sc as plsc

N = 100000
D = 128
V = 512
B = 128
NBLK = (N + B - 1) // B
MAXBASE = N - B
NC = 2
NS = 16
NW = NC * NS
ROUNDS = (NBLK + NW - 1) // NW
LANES = 16
CHUNKS = D // LANES
VSLICE = V // NS
XSLOTS = 3
GSLOTS = 2


def _body(x_hbm, din_hbm, dout_hbm, tin_hbm, tout_hbm, out_hbm,
          tin_sp, tout_sp,
          xb0, xb1, xb2, ab0, bb0, ii0, io0, ab1, bb1, ii1, io1,
          xs0, xs1, xs2, os0, os1, os2, gs0, gs1, is0, is1):
    cid = lax.axis_index("c")
    sid = lax.axis_index("s")
    w = sid * NC + cid

    def stage_table(t_hbm, t_sp, tmp, sem):
        rows = pl.ds(sid * VSLICE, VSLICE)
        pltpu.async_copy(t_hbm.at[rows], tmp, sem).wait()
        pltpu.sync_copy(tmp, t_sp.at[rows])

    stage_table(tin_hbm, tin_sp, ab0.at[pl.ds(0, VSLICE)], gs0)
    stage_table(tout_hbm, tout_sp, bb0.at[pl.ds(0, VSLICE)], gs0)
    plsc.subcore_barrier()

    xslot = ((xb0, xs0, os0), (xb1, xs1, os1), (xb2, xs2, os2))
    gslot = ((ab0, bb0, ii0, io0, gs0, is0), (ab1, bb1, ii1, io1, gs1, is1))

    def active(r):
        return (r >= 0) & (r < ROUNDS) & (r * NW + w < NBLK)

    def block_base(r):
        return jnp.minimum((r * NW + w) * B, MAXBASE)

    def emit_round(r, rx, rg):
        xb, xsem, osem = xslot[rx % XSLOTS]
        xbn, xsemn, osemn = xslot[(rx + 1) % XSLOTS]
        ab, bb, ii, io, gsem, isem = gslot[rg % GSLOTS]
        abn, bbn, iin, ion, gsemn, isemn = gslot[(rg + 1) % GSLOTS]

        @pl.when(active(r - 2))
        def _():
            pltpu.make_async_copy(xbn, out_hbm.at[pl.ds(0, B)], osemn).wait()

        @pl.when(active(r + 1))
        def _():
            pltpu.async_copy(x_hbm.at[pl.ds(block_base(r + 1), B)], xbn, xsemn)

        @pl.when(active(r + 1))
        def _():
            pltpu.make_async_copy(din_hbm.at[pl.ds(0, B)], iin, isemn).wait()
            pltpu.make_async_copy(dout_hbm.at[pl.ds(0, B)], ion, isemn).wait()
            pltpu.async_copy(tin_sp.at[iin], abn, gsemn)
            pltpu.async_copy(tout_sp.at[ion], bbn, gsemn)

        @pl.when(active(r))
        def _():
            pltpu.make_async_copy(x_hbm.at[pl.ds(0, B)], xb, xsem).wait()
            pltpu.make_async_copy(tin_sp.at[pl.ds(0, B)], ab, gsem).wait()
            pltpu.make_async_copy(tout_sp.at[pl.ds(0, B)], bb, gsem).wait()

        @pl.when(active(r + 2))
        def _():
            base2 = block_base(r + 2)
            pltpu.async_copy(din_hbm.at[pl.ds(base2, B)], ii, isem)
            pltpu.async_copy(dout_hbm.at[pl.ds(base2, B)], io, isem)

        @pl.when(active(r))
        def _():
            def row_body(i, c):
                for cc in range(CHUNKS):
                    sl = pl.ds(cc * LANES, LANES)
                    xb[i, sl] = xb[i, sl] + ab[i, sl] + bb[i, sl]
                return c

            lax.fori_loop(0, B, row_body, 0)
            pltpu.async_copy(xb, out_hbm.at[pl.ds(block_base(r), B)], osem)

    @pl.when(active(0))
    def _():
        pltpu.async_copy(din_hbm.at[pl.ds(block_base(0), B)], ii0, is0)
        pltpu.async_copy(dout_hbm.at[pl.ds(block_base(0), B)], io0, is0)

    @pl.when(active(1))
    def _():
        pltpu.async_copy(din_hbm.at[pl.ds(block_base(1), B)], ii1, is1)
        pltpu.async_copy(dout_hbm.at[pl.ds(block_base(1), B)], io1, is1)

    @pl.when(active(0))
    def _():
        pltpu.async_copy(x_hbm.at[pl.ds(block_base(0), B)], xb0, xs0)
        pltpu.make_async_copy(din_hbm.at[pl.ds(0, B)], ii0, is0).wait()
        pltpu.make_async_copy(dout_hbm.at[pl.ds(0, B)], io0, is0).wait()
        pltpu.async_copy(tin_sp.at[ii0], ab0, gs0)
        pltpu.async_copy(tout_sp.at[io0], bb0, gs0)

    def six_body(g, carry):
        for k in range(6):
            emit_round(6 * g + k, k, k)
        return carry

    lax.fori_loop(0, (ROUNDS + 2 + 5) // 6, six_body, 0)


@jax.jit
def kernel(x, in_degrees, out_degrees, in_degree_table, out_degree_table):
    mesh = plsc.VectorSubcoreMesh(
        core_axis_name="c", subcore_axis_name="s",
        num_cores=NC, num_subcores=NS,
    )
    xbuf = lambda: pltpu.VMEM((B, D), jnp.float32)
    ibuf = lambda: pltpu.VMEM((B,), jnp.int32)
    f = pl.kernel(
        _body,
        out_type=jax.ShapeDtypeStruct((N, D), jnp.float32),
        mesh=mesh,
        scratch_types=[
            pltpu.VMEM_SHARED((V, D), jnp.float32),
            pltpu.VMEM_SHARED((V, D), jnp.float32),
            xbuf(), xbuf(), xbuf(),
            xbuf(), xbuf(), ibuf(), ibuf(),
            xbuf(), xbuf(), ibuf(), ibuf(),
            pltpu.SemaphoreType.DMA, pltpu.SemaphoreType.DMA,
            pltpu.SemaphoreType.DMA, pltpu.SemaphoreType.DMA,
            pltpu.SemaphoreType.DMA, pltpu.SemaphoreType.DMA,
            pltpu.SemaphoreType.DMA, pltpu.SemaphoreType.DMA,
            pltpu.SemaphoreType.DMA, pltpu.SemaphoreType.DMA,
        ],
    )
    return f(x, in_degrees.astype(jnp.int32), out_degrees.astype(jnp.int32),
             in_degree_table, out_degree_table)
